# baseline (speedup 1.0000x reference)
import functools

import jax
import jax.numpy as jnp
from jax import lax
from jax.experimental import pallas as pl
from jax.experimental.pallas import tpu as pltpu
from jax.experimental.pallas import tpu_sc as plsc

_H = 16
_K = 22
_M = 8 * 512 * 512
_MP = _M // 2
_NC, _NS = 2, 16
_NW = _NC * _NS
_PER_W = _MP // _NW
_CH = 1024
_NCH = _PER_W // _CH

_mesh = plsc.VectorSubcoreMesh(core_axis_name="c", subcore_axis_name="s")


@functools.partial(
    pl.kernel,
    mesh=_mesh,
    compiler_params=pltpu.CompilerParams(use_tc_tiling_on_sc=False),
    out_type=jax.ShapeDtypeStruct((_MP, 2 * _H), jnp.float32),
    scratch_types=[
        pltpu.VMEM_SHARED((_K * _K, 2 * _H), jnp.float32),
        pltpu.VMEM((_CH,), jnp.int32),
        pltpu.VMEM((_CH, 2 * _H), jnp.float32),
        pltpu.SemaphoreType.DMA,
    ],
)
def _sc_lookup(pidx_hbm, ptable_hbm, out_hbm, table_v, idx_v, rows_v, sem):
    sid = lax.axis_index("s")
    wid = sid * _NC + lax.axis_index("c")
    base = wid * _PER_W

    @pl.when(sid == 0)
    def _load_table():
        pltpu.sync_copy(ptable_hbm, table_v)

    plsc.subcore_barrier()

    def chunk_body(c, carry):
        off = base + c * _CH
        pltpu.sync_copy(pidx_hbm.at[pl.ds(off, _CH)], idx_v)
        pltpu.async_copy(table_v.at[idx_v], rows_v, sem).wait()
        pltpu.sync_copy(rows_v, out_hbm.at[pl.ds(off, _CH)])
        return carry

    lax.fori_loop(0, _NCH, chunk_body, 0)


def kernel(dist, table):
    B, N, _ = dist.shape
    table_eff = table.at[0].set(0.0)
    d2 = dist.reshape(_MP, 2)
    pidx = d2[:, 0] * _K + d2[:, 1]
    ptable = jnp.concatenate(
        [
            jnp.repeat(table_eff, _K, axis=0),
            jnp.tile(table_eff, (_K, 1)),
        ],
        axis=1,
    )
    out = _sc_lookup(pidx, ptable)
    return out.reshape(B, N, N, _H)


# speedup vs baseline: 1.5034x; 1.5034x over previous
import functools

import jax
import jax.numpy as jnp
from jax import lax
from jax.experimental import pallas as pl
from jax.experimental.pallas import tpu as pltpu
from jax.experimental.pallas import tpu_sc as plsc

_H = 16
_K = 22
_M = 8 * 512 * 512
_MP = _M // 2
_NC, _NS = 2, 16
_NW = _NC * _NS
_PER_W = _MP // _NW
_CH = 1024
_NCH = _PER_W // _CH

_mesh = plsc.VectorSubcoreMesh(core_axis_name="c", subcore_axis_name="s")


@functools.partial(
    pl.kernel,
    mesh=_mesh,
    compiler_params=pltpu.CompilerParams(use_tc_tiling_on_sc=False),
    out_type=jax.ShapeDtypeStruct((_MP, 2 * _H), jnp.float32),
    scratch_types=[
        pltpu.VMEM_SHARED((_K * _K, 2 * _H), jnp.float32),
        pltpu.VMEM((_CH,), jnp.int32),
        pltpu.VMEM((_CH, 2 * _H), jnp.float32),
        pltpu.SemaphoreType.DMA,
    ],
)
def _sc_lookup(pidx_hbm, ptable_hbm, out_hbm, table_v, idx_v, rows_v, sem):
    sid = lax.axis_index("s")
    wid = sid * _NC + lax.axis_index("c")
    base = wid * _PER_W

    @pl.when(sid == 0)
    def _load_table():
        pltpu.sync_copy(ptable_hbm, table_v)

    plsc.subcore_barrier()

    def chunk_body(c, carry):
        off = base + c * _CH
        pltpu.sync_copy(pidx_hbm.at[pl.ds(off, _CH)], idx_v)
        pltpu.async_copy(table_v.at[idx_v], rows_v, sem).wait()
        pltpu.sync_copy(rows_v, out_hbm.at[pl.ds(off, _CH)])
        return carry

    lax.fori_loop(0, _NCH, chunk_body, 0)


def kernel(dist, table):
    B, N, _ = dist.shape
    table_eff = table.at[0].set(0.0)
    pidx = (dist[..., 0::2] * _K + dist[..., 1::2]).reshape(_MP)
    ptable = jnp.concatenate(
        [
            jnp.repeat(table_eff, _K, axis=0),
            jnp.tile(table_eff, (_K, 1)),
        ],
        axis=1,
    )
    out = _sc_lookup(pidx, ptable)
    return out.reshape(B, N, N, _H)


# speedup vs baseline: 1.9625x; 1.3054x over previous
import functools

import jax
import jax.numpy as jnp
from jax import lax
from jax.experimental import pallas as pl
from jax.experimental.pallas import tpu as pltpu
from jax.experimental.pallas import tpu_sc as plsc

_H = 16
_K = 22
_N = 512
_M = 8 * 512 * 512
_MP = _M // 2
_NC, _NS = 2, 16
_NW = _NC * _NS
_PER_W = _MP // _NW
_CH = 1024
_NCH = _PER_W // _CH
_RB = 128

_mesh = plsc.VectorSubcoreMesh(core_axis_name="c", subcore_axis_name="s")


def _pair_kernel(dist_ref, s_ref, out_ref):
    x = dist_ref[...].astype(jnp.float32)
    p = jnp.dot(x, s_ref[...], preferred_element_type=jnp.float32)
    out_ref[...] = p.astype(jnp.int32)


@functools.partial(
    pl.kernel,
    mesh=_mesh,
    compiler_params=pltpu.CompilerParams(use_tc_tiling_on_sc=False),
    out_type=jax.ShapeDtypeStruct((_MP, 2 * _H), jnp.float32),
    scratch_types=[
        pltpu.VMEM_SHARED((_K * _K, 2 * _H), jnp.float32),
        pltpu.VMEM((_CH,), jnp.int32),
        pltpu.VMEM((_CH, 2 * _H), jnp.float32),
        pltpu.SemaphoreType.DMA,
    ],
)
def _sc_lookup(pidx_hbm, ptable_hbm, out_hbm, table_v, idx_v, rows_v, sem):
    sid = lax.axis_index("s")
    wid = sid * _NC + lax.axis_index("c")
    base = wid * _PER_W

    @pl.when(sid == 0)
    def _load_table():
        pltpu.sync_copy(ptable_hbm, table_v)

    plsc.subcore_barrier()

    def chunk_body(c, carry):
        off = base + c * _CH
        pltpu.sync_copy(pidx_hbm.at[pl.ds(off, _CH)], idx_v)
        pltpu.async_copy(table_v.at[idx_v], rows_v, sem).wait()
        pltpu.sync_copy(rows_v, out_hbm.at[pl.ds(off, _CH)])
        return carry

    lax.fori_loop(0, _NCH, chunk_body, 0)


def kernel(dist, table):
    B, N, _ = dist.shape
    table_eff = table.at[0].set(0.0)

    j = jnp.arange(_N)[:, None]
    k = jnp.arange(_N // 2)[None, :]
    s = (22.0 * (j == 2 * k) + 1.0 * (j == 2 * k + 1)).astype(jnp.float32)

    d2 = dist.reshape(B * N, N)
    pidx = pl.pallas_call(
        _pair_kernel,
        grid=((B * N) // _RB,),
        in_specs=[
            pl.BlockSpec((_RB, _N), lambda i: (i, 0)),
            pl.BlockSpec((_N, _N // 2), lambda i: (0, 0)),
        ],
        out_specs=pl.BlockSpec((_RB, _N // 2), lambda i: (i, 0)),
        out_shape=jax.ShapeDtypeStruct((B * N, _N // 2), jnp.int32),
    )(d2, s)

    ptable = jnp.concatenate(
        [
            jnp.repeat(table_eff, _K, axis=0),
            jnp.tile(table_eff, (_K, 1)),
        ],
        axis=1,
    )
    out = _sc_lookup(pidx.reshape(_MP), ptable)
    return out.reshape(B, N, N, _H)


# speedup vs baseline: 2.0496x; 1.0444x over previous
import functools

import jax
import jax.numpy as jnp
from jax import lax
from jax.experimental import pallas as pl
from jax.experimental.pallas import tpu as pltpu
from jax.experimental.pallas import tpu_sc as plsc

_H = 16
_K = 22
_N = 512
_M = 8 * 512 * 512
_MP = _M // 2
_NC, _NS = 2, 16
_NW = _NC * _NS
_PER_W = _MP // _NW
_CH = 1024
_NCH = _PER_W // _CH
_RB = 128

_mesh = plsc.VectorSubcoreMesh(core_axis_name="c", subcore_axis_name="s")


def _pair_kernel(dist_ref, s_ref, out_ref):
    x = dist_ref[...].astype(jnp.float32)
    p = jnp.dot(x, s_ref[...], preferred_element_type=jnp.float32)
    out_ref[...] = p.astype(jnp.int32)


@functools.partial(
    pl.kernel,
    mesh=_mesh,
    compiler_params=pltpu.CompilerParams(use_tc_tiling_on_sc=False),
    out_type=jax.ShapeDtypeStruct((_MP, 2 * _H), jnp.float32),
    scratch_types=[
        pltpu.VMEM_SHARED((_K * _K, 2 * _H), jnp.float32),
        pltpu.VMEM((_CH,), jnp.int32),
        pltpu.VMEM((_CH,), jnp.int32),
        pltpu.VMEM((_CH, 2 * _H), jnp.float32),
        pltpu.VMEM((_CH, 2 * _H), jnp.float32),
        pltpu.SemaphoreType.DMA,
        pltpu.SemaphoreType.DMA,
        pltpu.SemaphoreType.DMA,
    ],
)
def _sc_lookup(pidx_hbm, ptable_hbm, out_hbm, table_v, idx_v0, idx_v1,
               rows_v0, rows_v1, semg, semo0, semo1):
    sid = lax.axis_index("s")
    wid = sid * _NC + lax.axis_index("c")
    base = wid * _PER_W

    @pl.when(sid == 0)
    def _load_table():
        pltpu.sync_copy(ptable_hbm, table_v)

    plsc.subcore_barrier()

    def half_body(cc, idx_v, rows_v, semo, first):
        off = base + cc * _CH
        pltpu.sync_copy(pidx_hbm.at[pl.ds(off, _CH)], idx_v)

        @pl.when(jnp.logical_not(first))
        def _wait_prev_out():
            pltpu.make_async_copy(
                rows_v, out_hbm.at[pl.ds(off, _CH)], semo).wait()

        pltpu.async_copy(table_v.at[idx_v], rows_v, semg).wait()
        pltpu.async_copy(rows_v, out_hbm.at[pl.ds(off, _CH)], semo)

    def chunk_body(c2, carry):
        first = c2 == 0
        half_body(c2 * 2, idx_v0, rows_v0, semo0, first)
        half_body(c2 * 2 + 1, idx_v1, rows_v1, semo1, first)
        return carry

    lax.fori_loop(0, _NCH // 2, chunk_body, 0)
    pltpu.make_async_copy(
        rows_v0, out_hbm.at[pl.ds(base, _CH)], semo0).wait()
    pltpu.make_async_copy(
        rows_v1, out_hbm.at[pl.ds(base, _CH)], semo1).wait()


def kernel(dist, table):
    B, N, _ = dist.shape
    table_eff = table.at[0].set(0.0)

    j = jnp.arange(_N)[:, None]
    k = jnp.arange(_N // 2)[None, :]
    s = (22.0 * (j == 2 * k) + 1.0 * (j == 2 * k + 1)).astype(jnp.float32)

    d2 = dist.reshape(B * N, N)
    pidx = pl.pallas_call(
        _pair_kernel,
        grid=((B * N) // _RB,),
        in_specs=[
            pl.BlockSpec((_RB, _N), lambda i: (i, 0)),
            pl.BlockSpec((_N, _N // 2), lambda i: (0, 0)),
        ],
        out_specs=pl.BlockSpec((_RB, _N // 2), lambda i: (i, 0)),
        out_shape=jax.ShapeDtypeStruct((B * N, _N // 2), jnp.int32),
    )(d2, s)

    ptable = jnp.concatenate(
        [
            jnp.repeat(table_eff, _K, axis=0),
            jnp.tile(table_eff, (_K, 1)),
        ],
        axis=1,
    )
    out = _sc_lookup(pidx.reshape(_MP), ptable)
    return out.reshape(B, N, N, _H)


# speedup vs baseline: 2.0843x; 1.0169x over previous
import functools

import jax
import jax.numpy as jnp
from jax import lax
from jax.experimental import pallas as pl
from jax.experimental.pallas import tpu as pltpu
from jax.experimental.pallas import tpu_sc as plsc

_H = 16
_K = 22
_N = 512
_M = 8 * 512 * 512
_MP = _M // 2
_NC, _NS = 2, 16
_NW = _NC * _NS
_PER_W = _MP // _NW
_CH = 1024
_NCH = _PER_W // _CH
_RB = 128

_mesh = plsc.VectorSubcoreMesh(core_axis_name="c", subcore_axis_name="s")


def _pair_kernel(dist_ref, s_ref, out_ref):
    x = dist_ref[...].astype(jnp.float32)
    p = jnp.dot(x, s_ref[...], preferred_element_type=jnp.float32)
    out_ref[...] = p.astype(jnp.int32)


@functools.partial(
    pl.kernel,
    mesh=_mesh,
    compiler_params=pltpu.CompilerParams(use_tc_tiling_on_sc=False),
    out_type=jax.ShapeDtypeStruct((_MP, 2 * _H), jnp.float32),
    scratch_types=[
        pltpu.VMEM_SHARED((_K * _K, 2 * _H), jnp.float32),
        pltpu.VMEM((_CH,), jnp.int32),
        pltpu.VMEM((_CH,), jnp.int32),
        pltpu.VMEM((_CH, 2 * _H), jnp.float32),
        pltpu.VMEM((_CH, 2 * _H), jnp.float32),
        pltpu.SemaphoreType.DMA,
        pltpu.SemaphoreType.DMA,
        pltpu.SemaphoreType.DMA,
        pltpu.SemaphoreType.DMA,
        pltpu.SemaphoreType.DMA,
    ],
)
def _sc_lookup(pidx_hbm, ptable_hbm, out_hbm, table_v, idx_v0, idx_v1,
               rows_v0, rows_v1, semg, semi0, semi1, semo0, semo1):
    sid = lax.axis_index("s")
    wid = sid * _NC + lax.axis_index("c")
    base = wid * _PER_W

    @pl.when(sid == 0)
    def _load_table():
        pltpu.sync_copy(ptable_hbm, table_v)

    plsc.subcore_barrier()

    pltpu.async_copy(pidx_hbm.at[pl.ds(base, _CH)], idx_v0, semi0)
    pltpu.async_copy(pidx_hbm.at[pl.ds(base + _CH, _CH)], idx_v1, semi1)

    def half_body(cc, idx_v, rows_v, semi, semo, first, last):
        off = base + cc * _CH
        pltpu.make_async_copy(
            pidx_hbm.at[pl.ds(off, _CH)], idx_v, semi).wait()

        @pl.when(jnp.logical_not(first))
        def _wait_prev_out():
            pltpu.make_async_copy(
                rows_v, out_hbm.at[pl.ds(off, _CH)], semo).wait()

        pltpu.async_copy(table_v.at[idx_v], rows_v, semg).wait()

        @pl.when(jnp.logical_not(last))
        def _prefetch_next_idx():
            pltpu.async_copy(
                pidx_hbm.at[pl.ds(off + 2 * _CH, _CH)], idx_v, semi)

        pltpu.async_copy(rows_v, out_hbm.at[pl.ds(off, _CH)], semo)

    def chunk_body(c2, carry):
        first = c2 == 0
        last = c2 == (_NCH // 2 - 1)
        half_body(c2 * 2, idx_v0, rows_v0, semi0, semo0, first, last)
        half_body(c2 * 2 + 1, idx_v1, rows_v1, semi1, semo1, first, last)
        return carry

    lax.fori_loop(0, _NCH // 2, chunk_body, 0)
    pltpu.make_async_copy(
        rows_v0, out_hbm.at[pl.ds(base, _CH)], semo0).wait()
    pltpu.make_async_copy(
        rows_v1, out_hbm.at[pl.ds(base, _CH)], semo1).wait()


def kernel(dist, table):
    B, N, _ = dist.shape
    table_eff = table.at[0].set(0.0)

    j = jnp.arange(_N)[:, None]
    k = jnp.arange(_N // 2)[None, :]
    s = (22.0 * (j == 2 * k) + 1.0 * (j == 2 * k + 1)).astype(jnp.float32)

    d2 = dist.reshape(B * N, N)
    pidx = pl.pallas_call(
        _pair_kernel,
        grid=((B * N) // _RB,),
        in_specs=[
            pl.BlockSpec((_RB, _N), lambda i: (i, 0)),
            pl.BlockSpec((_N, _N // 2), lambda i: (0, 0)),
        ],
        out_specs=pl.BlockSpec((_RB, _N // 2), lambda i: (i, 0)),
        out_shape=jax.ShapeDtypeStruct((B * N, _N // 2), jnp.int32),
    )(d2, s)

    ptable = jnp.concatenate(
        [
            jnp.repeat(table_eff, _K, axis=0),
            jnp.tile(table_eff, (_K, 1)),
        ],
        axis=1,
    )
    out = _sc_lookup(pidx.reshape(_MP), ptable)
    return out.reshape(B, N, N, _H)
